# Initial kernel scaffold; baseline (speedup 1.0000x reference)
import functools

import jax
import jax.numpy as jnp
from jax import lax
from jax.experimental import pallas as pl
from jax.experimental.pallas import tpu as pltpu
from jax.experimental.pallas import tpu_sc as plsc

N_NODES = 10000
N_EDGES = 320000
D_FEAT = 128

NC = 2
NS = 16
NW = NC * NS

CHUNK = 128
N_ROWS = N_EDGES // CHUNK
ROWS_PER_SC = N_NODES // NS


def _sc_accumulate(x_hbm, src_hbm, dst_hbm, part_hbm,
                   acc_sh, src_v, dst_v, rows_v, zero_v, gsem):
    c = lax.axis_index("c")
    s = lax.axis_index("s")
    wid = s * NC + c

    def _zero_vmem(i, _):
        for j in range(8):
            zero_v[i, pl.ds(j * 16, 16)] = jnp.zeros((16,), jnp.float32)
        return 0
    lax.fori_loop(0, 125, _zero_vmem, 0)
    zbase = s * ROWS_PER_SC
    for r in range(5):
        pltpu.sync_copy(zero_v, acc_sh.at[pl.ds(zbase + r * 125, 125), :])
    plsc.subcore_barrier()

    nrows = jnp.where(wid < 4, 79, 78)
    rbase = wid * 78 + jnp.minimum(wid, 4)

    def _edge_step(i, _):
        row = rbase + i
        pltpu.sync_copy(src_hbm.at[row, :], src_v)
        pltpu.sync_copy(dst_hbm.at[row, :], dst_v)
        pltpu.async_copy(x_hbm.at[src_v], rows_v, gsem).wait()
        pltpu.sync_copy(rows_v, acc_sh.at[dst_v], add=True)
        return 0
    lax.fori_loop(0, nrows, _edge_step, 0)
    plsc.subcore_barrier()

    wbase = s * ROWS_PER_SC
    for r in range(5):
        pltpu.sync_copy(acc_sh.at[pl.ds(wbase + r * 125, 125), :],
                        part_hbm.at[c, pl.ds(wbase + r * 125, 125), :])


def _combine_body(p_ref, o_ref):
    o_ref[...] = p_ref[0] + p_ref[1]


@jax.jit
def kernel(x, edge_index):
    src2d = edge_index[0].reshape(N_ROWS, CHUNK)
    dst2d = edge_index[1].reshape(N_ROWS, CHUNK)

    mesh = plsc.VectorSubcoreMesh(core_axis_name="c", subcore_axis_name="s",
                                  num_cores=NC, num_subcores=NS)
    partials = pl.kernel(
        _sc_accumulate,
        out_type=jax.ShapeDtypeStruct((NC, N_NODES, D_FEAT), jnp.float32),
        mesh=mesh,
        scratch_types=[
            pltpu.VMEM_SHARED((N_NODES, D_FEAT), jnp.float32),
            pltpu.VMEM((CHUNK,), jnp.int32),
            pltpu.VMEM((CHUNK,), jnp.int32),
            pltpu.VMEM((CHUNK, D_FEAT), jnp.float32),
            pltpu.VMEM((125, D_FEAT), jnp.float32),
            pltpu.SemaphoreType.DMA,
        ],
    )(x, src2d, dst2d)

    out = pl.pallas_call(
        _combine_body,
        out_shape=jax.ShapeDtypeStruct((N_NODES, D_FEAT), jnp.float32),
        grid=(10,),
        in_specs=[pl.BlockSpec((NC, N_NODES // 10, D_FEAT),
                               lambda i: (0, i, 0))],
        out_specs=pl.BlockSpec((N_NODES // 10, D_FEAT), lambda i: (i, 0)),
    )(partials)
    return out


# speedup vs baseline: 6.8080x; 6.8080x over previous
import functools

import jax
import jax.numpy as jnp
from jax import lax
from jax.experimental import pallas as pl
from jax.experimental.pallas import tpu as pltpu
from jax.experimental.pallas import tpu_sc as plsc

N_NODES = 10000
N_EDGES = 320000
D_FEAT = 128

NC = 2
NS = 16
NW = NC * NS

CHUNK = 128
N_ROWS = N_EDGES // CHUNK
ZROWS = 624


def _sc_accumulate(x_hbm, src_hbm, dst_hbm, part_hbm,
                   acc_sh, src_v, dst_v, rows_v, gsem):
    c = lax.axis_index("c")
    s = lax.axis_index("s")
    wid = s * NC + c

    def _zero_vmem(i, _):
        for j in range(8):
            rows_v[i, pl.ds(j * 16, 16)] = jnp.zeros((16,), jnp.float32)
        return 0
    lax.fori_loop(0, CHUNK, _zero_vmem, 0)
    zbase = s * ZROWS
    for k in range(4):
        pltpu.sync_copy(rows_v, acc_sh.at[pl.ds(zbase + k * CHUNK, CHUNK), :])
    pltpu.sync_copy(rows_v.at[pl.ds(0, 112), :],
                    acc_sh.at[pl.ds(zbase + 4 * CHUNK, 112), :])

    @pl.when(s == 0)
    def _zero_tail():
        pltpu.sync_copy(rows_v.at[pl.ds(0, 16), :],
                        acc_sh.at[pl.ds(NS * ZROWS, 16), :])
    plsc.subcore_barrier()

    nrows = jnp.where(wid < 4, 79, 78)
    rbase = wid * 78 + jnp.minimum(wid, 4)

    def _edge_step(i, _):
        row = rbase + i
        pltpu.sync_copy(src_hbm.at[row, :], src_v)
        pltpu.sync_copy(dst_hbm.at[row, :], dst_v)
        pltpu.async_copy(x_hbm.at[src_v], rows_v, gsem).wait()
        pltpu.sync_copy(rows_v, acc_sh.at[dst_v], add=True)
        return 0
    lax.fori_loop(0, nrows, _edge_step, 0)
    plsc.subcore_barrier()

    wbase = s * ZROWS
    pltpu.sync_copy(acc_sh.at[pl.ds(wbase, ZROWS), :],
                    part_hbm.at[c, pl.ds(wbase, ZROWS), :])

    @pl.when(s == 0)
    def _write_tail():
        pltpu.sync_copy(acc_sh.at[pl.ds(NS * ZROWS, 16), :],
                        part_hbm.at[c, pl.ds(NS * ZROWS, 16), :])


def _combine_body(p_ref, o_ref):
    o_ref[...] = p_ref[0] + p_ref[1]


@jax.jit
def kernel(x, edge_index):
    src2d = edge_index[0].reshape(N_ROWS, CHUNK)
    dst2d = edge_index[1].reshape(N_ROWS, CHUNK)

    mesh = plsc.VectorSubcoreMesh(core_axis_name="c", subcore_axis_name="s",
                                  num_cores=NC, num_subcores=NS)
    partials = pl.kernel(
        _sc_accumulate,
        out_type=jax.ShapeDtypeStruct((NC, N_NODES, D_FEAT), jnp.float32),
        mesh=mesh,
        scratch_types=[
            pltpu.VMEM_SHARED((N_NODES, D_FEAT), jnp.float32),
            pltpu.VMEM((CHUNK,), jnp.int32),
            pltpu.VMEM((CHUNK,), jnp.int32),
            pltpu.VMEM((CHUNK, D_FEAT), jnp.float32),
            pltpu.SemaphoreType.DMA,
        ],
    )(x, src2d, dst2d)

    out = pl.pallas_call(
        _combine_body,
        out_shape=jax.ShapeDtypeStruct((N_NODES, D_FEAT), jnp.float32),
        grid=(10,),
        in_specs=[pl.BlockSpec((NC, N_NODES // 10, D_FEAT),
                               lambda i: (0, i, 0))],
        out_specs=pl.BlockSpec((N_NODES // 10, D_FEAT), lambda i: (i, 0)),
    )(partials)
    return out
